# baseline (speedup 1.0000x reference)
import functools

import jax
import jax.numpy as jnp
from jax import lax
from jax.experimental import pallas as pl
from jax.experimental.pallas import tpu as pltpu
from jax.experimental.pallas import tpu_sc as plsc

N = 10000
D = 128
H = 128
OUT_A = 64
E = 320000

NC = 2
NS = 16
NW = NC * NS

C = 128
ECH = 80
EPW = ECH * C
E_PAD = EPW * NW

RCH = 80
R = RCH * C

_mesh = plsc.VectorSubcoreMesh(core_axis_name="c", subcore_axis_name="s")


CH0 = 89
CH1 = 71
NBUF = 3

RSCH = 79
RS = RSCH * C


def _spmm_one(x_hbm, src_hbm, dst_hbm, zeros_hbm, out_hbm, slot, acc,
              sidx, didx, rows, s_si, s_di, s_g, s_s):
    c = lax.axis_index("c")
    s = lax.axis_index("s")
    pltpu.sync_copy(zeros_hbm, rows[2])
    for j in range(5):
        ch = s * 5 + j
        @pl.when(ch < RSCH)
        def _():
            pltpu.sync_copy(rows[2], acc.at[pl.ds(ch * C, C)])
    plsc.subcore_barrier()
    kc = lax.select(c == 0, jnp.int32(CH0), jnp.int32(CH1))
    base = (c * NS * CH0 + s * kc) * C

    def issue_idx(k, b):
        off = base + k * C
        pltpu.async_copy(src_hbm.at[pl.ds(off, C)], sidx[b], s_si[b])
        pltpu.async_copy(dst_hbm.at[pl.ds(off, C)], didx[b], s_di[b])

    def wait_idx(k, b):
        off = base + k * C
        pltpu.make_async_copy(src_hbm.at[pl.ds(off, C)], sidx[b], s_si[b]).wait()
        pltpu.make_async_copy(dst_hbm.at[pl.ds(off, C)], didx[b], s_di[b]).wait()

    def issue_gather(b):
        pltpu.async_copy(x_hbm.at[sidx[b]], rows[b], s_g[b])

    def wait_gather(b):
        pltpu.make_async_copy(x_hbm.at[sidx[b]], rows[b], s_g[b]).wait()

    def issue_scatter(b):
        pltpu.async_copy(rows[b], acc.at[didx[b]], s_s[b], add=True)

    def wait_scatter(b):
        pltpu.make_async_copy(rows[b], acc.at[didx[b]], s_s[b]).wait()

    issue_idx(0, 0)
    wait_idx(0, 0)
    issue_gather(0)
    issue_idx(1, 1)
    wait_idx(1, 1)
    issue_gather(1)
    issue_idx(2, 2)
    wait_gather(0)
    issue_scatter(0)

    def triple(t, carry):
        for u in range(NBUF):
            k = 2 + NBUF * t + u
            b = (2 + u) % NBUF
            wait_idx(k, b)
            wait_scatter(u)
            issue_gather(b)
            @pl.when(k + 1 < kc)
            def _():
                issue_idx(k + 1, u)
            wait_gather((u + 1) % NBUF)
            issue_scatter((u + 1) % NBUF)
        return carry

    lax.fori_loop(0, (kc - 2) // NBUF, triple, 0)
    wait_scatter(0)
    wait_gather(1)
    issue_scatter(1)
    wait_scatter(1)
    plsc.subcore_barrier()
    for j in range(5):
        ch = s * 5 + j
        @pl.when(ch < RSCH)
        def _():
            pltpu.sync_copy(acc.at[pl.ds(ch * C, C)], rows[0])
            pltpu.sync_copy(rows[0], out_hbm.at[slot, c, pl.ds(ch * C, C)])


def _spmm2_body(x1_hbm, x2_hbm, src1_hbm, dst1_hbm, src2_hbm, dst2_hbm,
                zeros_hbm, out_hbm, acc, *bufs):
    sidx = bufs[0:3]
    didx = bufs[3:6]
    rows = bufs[6:9]
    s_si = bufs[9:12]
    s_di = bufs[12:15]
    s_g = bufs[15:18]
    s_s = bufs[18:21]
    _spmm_one(x1_hbm, src1_hbm, dst1_hbm, zeros_hbm, out_hbm, 0, acc,
              sidx, didx, rows, s_si, s_di, s_g, s_s)
    _spmm_one(x2_hbm, src2_hbm, dst2_hbm, zeros_hbm, out_hbm, 1, acc,
              sidx, didx, rows, s_si, s_di, s_g, s_s)


_spmm2 = pl.kernel(
    _spmm2_body,
    out_type=jax.ShapeDtypeStruct((2, NC, RS, D), jnp.float32),
    mesh=_mesh,
    scratch_types=(
        [pltpu.VMEM_SHARED((RS, D), jnp.float32)]
        + [pltpu.VMEM((C,), jnp.int32) for _ in range(6)]
        + [pltpu.VMEM((C, D), jnp.float32) for _ in range(3)]
        + [pltpu.SemaphoreType.DMA for _ in range(12)]
    ),
)


DCHK = 1280
DGRP = DCHK // 16
HR = 80
AR = 4 * HR
UNR = 512


def _deg_body(idx_hbm, zeros_hbm, ar_hbm, out_hbm, acc, hist, idxv, pk, unp,
              ar0, ar1, ar2):
    c = lax.axis_index("c")
    s = lax.axis_index("s")
    wid = s * NC + c
    pltpu.sync_copy(zeros_hbm, hist.at[pl.ds(0, C)])
    pltpu.sync_copy(zeros_hbm, hist.at[pl.ds(C, C)])
    pltpu.sync_copy(zeros_hbm.at[pl.ds(0, AR - 2 * C)],
                    hist.at[pl.ds(2 * C, AR - 2 * C)])
    @pl.when(s < 8)
    def _():
        pltpu.sync_copy(zeros_hbm.at[pl.ds(0, 40)], acc.at[pl.ds(s * 40, 40)])
    pltpu.sync_copy(ar_hbm.at[pl.ds(0, C)], ar0)
    pltpu.sync_copy(ar_hbm.at[pl.ds(C, C)], ar1)
    pltpu.sync_copy(ar_hbm.at[pl.ds(2 * C, AR - 2 * C)], ar2)
    ones = jnp.full((16,), 1.0, jnp.float32)
    for k in range(4):
        base = k * E_PAD + wid * EPW
        def chunk(i, carry):
            pltpu.sync_copy(idx_hbm.at[pl.ds(base + i * DCHK, DCHK)], idxv)
            def grp(g, carry2):
                iv = idxv[pl.ds(g * 16, 16)]
                row = lax.shift_right_logical(iv, 7) + k * HR
                col = lax.bitwise_and(iv, 127)
                plsc.addupdate_scatter(hist, [row, col], ones)
                return carry2
            lax.fori_loop(0, DGRP, grp, 0)
            return carry
        lax.fori_loop(0, 8, chunk, 0)
    plsc.subcore_barrier()
    pltpu.sync_copy(hist.at[pl.ds(0, C)], acc.at[ar0], add=True)
    pltpu.sync_copy(hist.at[pl.ds(C, C)], acc.at[ar1], add=True)
    pltpu.sync_copy(hist.at[pl.ds(2 * C, AR - 2 * C)], acc.at[ar2], add=True)
    plsc.subcore_barrier()
    @pl.when(s < 10)
    def _():
        m = s
        for k in range(4):
            pltpu.sync_copy(acc.at[pl.ds(k * HR + 8 * m, 8)], pk.at[k])
        for h in range(2):
            for k in range(4):
                for r in range(4):
                    rr = 4 * h + r
                    for g in range(8):
                        v = pk[k, rr, pl.ds(g * 16, 16)]
                        orow = lax.iota(jnp.int32, 16) + (r * C + g * 16)
                        plsc.store_scatter(
                            unp, [orow, jnp.full((16,), k, jnp.int32)], v)
            pltpu.sync_copy(unp, out_hbm.at[c, pl.ds(1024 * m + UNR * h, UNR)])


_deg = pl.kernel(
    _deg_body,
    out_type=jax.ShapeDtypeStruct((NC, R, D), jnp.float32),
    mesh=_mesh,
    compiler_params=pltpu.CompilerParams(needs_layout_passes=False),
    scratch_types=[
        pltpu.VMEM_SHARED((AR, D), jnp.float32),
        pltpu.VMEM((AR, D), jnp.float32),
        pltpu.VMEM((DCHK,), jnp.int32),
        pltpu.VMEM((4, 8, D), jnp.float32),
        pltpu.VMEM((UNR, D), jnp.float32),
        pltpu.VMEM((C,), jnp.int32),
        pltpu.VMEM((C,), jnp.int32),
        pltpu.VMEM((AR - 2 * C,), jnp.int32),
    ],
)


BR = 400


def _embed_body(x_ref, w_ref, b_ref, dp_ref, o_ref, *, lane):
    h = jnp.dot(x_ref[...], w_ref[...], preferred_element_type=jnp.float32)
    h = h + b_ref[...]
    deg = dp_ref[0, :, lane:lane + 1] + dp_ref[1, :, lane:lane + 1]
    o_ref[...] = h * lax.rsqrt(jnp.maximum(deg, 1.0))


def _embed(x, w, b, dp, lane):
    return pl.pallas_call(
        functools.partial(_embed_body, lane=lane),
        grid=(N // BR,),
        in_specs=[
            pl.BlockSpec((BR, D), lambda i: (i, 0)),
            pl.BlockSpec((D, H), lambda i: (0, 0)),
            pl.BlockSpec((1, H), lambda i: (0, 0)),
            pl.BlockSpec((NC, BR, D), lambda i: (0, i, 0)),
        ],
        out_specs=pl.BlockSpec((BR, H), lambda i: (i, 0)),
        out_shape=jax.ShapeDtypeStruct((N, H), jnp.float32),
    )(x, w, b, dp)


def _comb_body(p_ref, d_ref, o_ref, *, lane_a, lane_b):
    t = p_ref[0, 0] + p_ref[0, 1]
    da = d_ref[0, :, lane_a:lane_a + 1] + d_ref[1, :, lane_a:lane_a + 1]
    db = d_ref[0, :, lane_b:lane_b + 1] + d_ref[1, :, lane_b:lane_b + 1]
    sa = lax.rsqrt(jnp.maximum(da, 1.0))
    sb = lax.rsqrt(jnp.maximum(db, 1.0))
    o_ref[...] = t * (sa * sb)


def _comb(p, slot, d, lane_a, lane_b):
    return pl.pallas_call(
        functools.partial(_comb_body, lane_a=lane_a, lane_b=lane_b),
        grid=(N // BR,),
        in_specs=[
            pl.BlockSpec((1, NC, BR, H), lambda i: (slot, 0, i, 0)),
            pl.BlockSpec((NC, BR, D), lambda i: (0, i, 0)),
        ],
        out_specs=pl.BlockSpec((BR, H), lambda i: (i, 0)),
        out_shape=jax.ShapeDtypeStruct((N, H), jnp.float32),
    )(p, d)


def _head_body(p_ref, d_ref, g_ref, bb_ref, w1_ref, b1_ref, w2_ref, b2_ref,
               o_ref, *, softmax, lane):
    h = p_ref[0, 0] + p_ref[0, 1]
    deg = d_ref[0, :, lane:lane + 1] + d_ref[1, :, lane:lane + 1]
    h = h * lax.rsqrt(jnp.maximum(deg, 1.0))
    mu = jnp.mean(h, axis=-1, keepdims=True)
    xc = h - mu
    var = jnp.mean(xc * xc, axis=-1, keepdims=True)
    hn = xc * lax.rsqrt(var + 1e-5) * g_ref[...] + bb_ref[...]
    z = jnp.maximum(
        jnp.dot(hn, w1_ref[...], preferred_element_type=jnp.float32)
        + b1_ref[...], 0.0)
    z = jnp.dot(z, w2_ref[...], preferred_element_type=jnp.float32) + b2_ref[...]
    if softmax:
        m = jnp.max(z, axis=-1, keepdims=True)
        z = z - m
        z = z - jnp.log(jnp.sum(jnp.exp(z), axis=-1, keepdims=True))
    o_ref[...] = z


def _head(p, slot, d, g, b, w1, b1, w2, b2, odim, softmax, lane):
    return pl.pallas_call(
        functools.partial(_head_body, softmax=softmax, lane=lane),
        grid=(N // BR,),
        in_specs=[
            pl.BlockSpec((1, NC, BR, H), lambda i: (slot, 0, i, 0)),
            pl.BlockSpec((NC, BR, D), lambda i: (0, i, 0)),
            pl.BlockSpec((1, H), lambda i: (0, 0)),
            pl.BlockSpec((1, H), lambda i: (0, 0)),
            pl.BlockSpec((H, H), lambda i: (0, 0)),
            pl.BlockSpec((1, H), lambda i: (0, 0)),
            pl.BlockSpec((H, odim), lambda i: (0, 0)),
            pl.BlockSpec((1, odim), lambda i: (0, 0)),
        ],
        out_specs=pl.BlockSpec((BR, odim), lambda i: (i, 0)),
        out_shape=jax.ShapeDtypeStruct((N, odim), jnp.float32),
    )(p, d, g, b, w1, b1, w2, b2)


def kernel(x_author, x_paper, edge_writes, edge_rev, W_emb_a, b_emb_a,
           W_emb_p, b_emb_p, ln_g_a, ln_b_a, ln_g_p, ln_b_p, Wo1_a, bo1_a,
           Wo2_a, bo2_a, Wo1_p, bo1_p, Wo2_p, bo2_p):
    ew = edge_writes.astype(jnp.int32)
    er = edge_rev.astype(jnp.int32)
    pad = E_PAD - E

    def pad_to(a, val):
        return jnp.concatenate([a, jnp.full((pad,), val, jnp.int32)])

    pad_dst = N + jnp.arange(pad, dtype=jnp.int32) % (RS - N)
    pad_src = jnp.arange(pad, dtype=jnp.int32) % N

    src_w = jnp.concatenate([ew[0], pad_src])
    dst_w = jnp.concatenate([ew[1], pad_dst])
    src_r = jnp.concatenate([er[0], pad_src])
    dst_r = jnp.concatenate([er[1], pad_dst])
    deg_idx = jnp.concatenate([
        pad_to(ew[0], N), pad_to(ew[1], N),
        pad_to(er[0], N), pad_to(er[1], N),
    ])
    zeros128 = jnp.zeros((C, D), jnp.float32)
    arange320 = jnp.arange(AR, dtype=jnp.int32)

    degp = _deg(deg_idx, zeros128, arange320)

    a0 = _embed(x_author, W_emb_a, b_emb_a.reshape(1, H), degp, 0)
    p0 = _embed(x_paper, W_emb_p, b_emb_p.reshape(1, H), degp, 2)

    t01 = _spmm2(a0, p0, src_w, dst_w, src_r, dst_r, zeros128)

    a1 = _comb(t01, 0, degp, 1, 2)
    p1 = _comb(t01, 1, degp, 3, 0)

    h01 = _spmm2(a1, p1, src_r, dst_r, src_w, dst_w, zeros128)

    out_a = _head(h01, 0, degp, ln_g_a.reshape(1, H), ln_b_a.reshape(1, H),
                  Wo1_a, bo1_a.reshape(1, H), Wo2_a, bo2_a.reshape(1, OUT_A),
                  OUT_A, True, 3)
    out_p = _head(h01, 1, degp, ln_g_p.reshape(1, H), ln_b_p.reshape(1, H),
                  Wo1_p, bo1_p.reshape(1, H), Wo2_p, bo2_p.reshape(1, H),
                  H, False, 1)
    return (out_a, out_p)


# speedup vs baseline: 1.1313x; 1.1313x over previous
import functools

import jax
import jax.numpy as jnp
from jax import lax
from jax.experimental import pallas as pl
from jax.experimental.pallas import tpu as pltpu
from jax.experimental.pallas import tpu_sc as plsc

N = 10000
D = 128
H = 128
OUT_A = 64
E = 320000

NC = 2
NS = 16
NW = NC * NS

C = 128
ECH = 80
EPW = ECH * C
E_PAD = EPW * NW

RCH = 80
R = RCH * C

_mesh = plsc.VectorSubcoreMesh(core_axis_name="c", subcore_axis_name="s")


CH0 = 83
CH1 = 77
NBUF = 3

RSCH = 79
RS = RSCH * C


def _spmm_one(x_hbm, src_hbm, dst_hbm, zeros_hbm, out_hbm, slot, acc,
              sidx, didx, rows, s_si, s_di, s_g, s_s):
    c = lax.axis_index("c")
    s = lax.axis_index("s")
    pltpu.sync_copy(zeros_hbm, rows[2])
    for j in range(5):
        ch = s * 5 + j
        @pl.when(ch < RSCH)
        def _():
            pltpu.sync_copy(rows[2], acc.at[pl.ds(ch * C, C)])
    plsc.subcore_barrier()
    kc = lax.select(c == 0, jnp.int32(CH0), jnp.int32(CH1))
    base = (c * NS * CH0 + s * kc) * C

    def issue_idx(k, b):
        off = base + k * C
        pltpu.async_copy(src_hbm.at[pl.ds(off, C)], sidx[b], s_si[b])
        pltpu.async_copy(dst_hbm.at[pl.ds(off, C)], didx[b], s_di[b])

    def wait_idx(k, b):
        off = base + k * C
        pltpu.make_async_copy(src_hbm.at[pl.ds(off, C)], sidx[b], s_si[b]).wait()
        pltpu.make_async_copy(dst_hbm.at[pl.ds(off, C)], didx[b], s_di[b]).wait()

    def issue_gather(b):
        pltpu.async_copy(x_hbm.at[sidx[b]], rows[b], s_g[b])

    def wait_gather(b):
        pltpu.make_async_copy(x_hbm.at[sidx[b]], rows[b], s_g[b]).wait()

    def issue_scatter(b):
        pltpu.async_copy(rows[b], acc.at[didx[b]], s_s[b], add=True)

    def wait_scatter(b):
        pltpu.make_async_copy(rows[b], acc.at[didx[b]], s_s[b]).wait()

    issue_idx(0, 0)
    wait_idx(0, 0)
    issue_gather(0)
    issue_idx(1, 1)
    wait_idx(1, 1)
    issue_gather(1)
    issue_idx(2, 2)
    wait_gather(0)
    issue_scatter(0)

    def triple(t, carry):
        for u in range(NBUF):
            k = 2 + NBUF * t + u
            b = (2 + u) % NBUF
            wait_idx(k, b)
            wait_scatter(u)
            issue_gather(b)
            @pl.when(k + 1 < kc)
            def _():
                issue_idx(k + 1, u)
            wait_gather((u + 1) % NBUF)
            issue_scatter((u + 1) % NBUF)
        return carry

    lax.fori_loop(0, (kc - 2) // NBUF, triple, 0)
    wait_scatter(0)
    wait_gather(1)
    issue_scatter(1)
    wait_scatter(1)
    plsc.subcore_barrier()
    for j in range(5):
        ch = s * 5 + j
        @pl.when(ch < RSCH)
        def _():
            pltpu.sync_copy(acc.at[pl.ds(ch * C, C)], rows[0])
            pltpu.sync_copy(rows[0], out_hbm.at[slot, c, pl.ds(ch * C, C)])


def _spmm_body(x_hbm, src_hbm, dst_hbm, zeros_hbm, out_hbm, acc, *bufs):
    sidx = bufs[0:3]
    didx = bufs[3:6]
    rows = bufs[6:9]
    s_si = bufs[9:12]
    s_di = bufs[12:15]
    s_g = bufs[15:18]
    s_s = bufs[18:21]
    _spmm_one(x_hbm, src_hbm, dst_hbm, zeros_hbm, out_hbm, 0, acc,
              sidx, didx, rows, s_si, s_di, s_g, s_s)


_spmm = pl.kernel(
    _spmm_body,
    out_type=jax.ShapeDtypeStruct((1, NC, RS, D), jnp.float32),
    mesh=_mesh,
    scratch_types=(
        [pltpu.VMEM_SHARED((RS, D), jnp.float32)]
        + [pltpu.VMEM((C,), jnp.int32) for _ in range(6)]
        + [pltpu.VMEM((C, D), jnp.float32) for _ in range(3)]
        + [pltpu.SemaphoreType.DMA for _ in range(12)]
    ),
)


DCHK = 1280
DGRP = DCHK // 16
HR = 80
AR = 4 * HR
UNR = 512


def _deg_body(idx_hbm, zeros_hbm, ar_hbm, out_hbm, acc, hist, idxv, pk, unp,
              ar0, ar1, ar2):
    c = lax.axis_index("c")
    s = lax.axis_index("s")
    wid = s * NC + c
    pltpu.sync_copy(zeros_hbm, hist.at[pl.ds(0, C)])
    pltpu.sync_copy(zeros_hbm, hist.at[pl.ds(C, C)])
    pltpu.sync_copy(zeros_hbm.at[pl.ds(0, AR - 2 * C)],
                    hist.at[pl.ds(2 * C, AR - 2 * C)])
    @pl.when(s < 8)
    def _():
        pltpu.sync_copy(zeros_hbm.at[pl.ds(0, 40)], acc.at[pl.ds(s * 40, 40)])
    pltpu.sync_copy(ar_hbm.at[pl.ds(0, C)], ar0)
    pltpu.sync_copy(ar_hbm.at[pl.ds(C, C)], ar1)
    pltpu.sync_copy(ar_hbm.at[pl.ds(2 * C, AR - 2 * C)], ar2)
    ones = jnp.full((16,), 1.0, jnp.float32)
    for k in range(4):
        base = k * E_PAD + wid * EPW
        def chunk(i, carry):
            pltpu.sync_copy(idx_hbm.at[pl.ds(base + i * DCHK, DCHK)], idxv)
            def grp(g, carry2):
                iv = idxv[pl.ds(g * 16, 16)]
                row = lax.shift_right_logical(iv, 7) + k * HR
                col = lax.bitwise_and(iv, 127)
                plsc.addupdate_scatter(hist, [row, col], ones)
                return carry2
            lax.fori_loop(0, DGRP, grp, 0)
            return carry
        lax.fori_loop(0, 8, chunk, 0)
    plsc.subcore_barrier()
    pltpu.sync_copy(hist.at[pl.ds(0, C)], acc.at[ar0], add=True)
    pltpu.sync_copy(hist.at[pl.ds(C, C)], acc.at[ar1], add=True)
    pltpu.sync_copy(hist.at[pl.ds(2 * C, AR - 2 * C)], acc.at[ar2], add=True)
    plsc.subcore_barrier()
    @pl.when(s < 10)
    def _():
        m = s
        for k in range(4):
            pltpu.sync_copy(acc.at[pl.ds(k * HR + 8 * m, 8)], pk.at[k])
        for h in range(2):
            for k in range(4):
                for r in range(4):
                    rr = 4 * h + r
                    for g in range(8):
                        v = pk[k, rr, pl.ds(g * 16, 16)]
                        orow = lax.iota(jnp.int32, 16) + (r * C + g * 16)
                        plsc.store_scatter(
                            unp, [orow, jnp.full((16,), k, jnp.int32)], v)
            pltpu.sync_copy(unp, out_hbm.at[c, pl.ds(1024 * m + UNR * h, UNR)])


_deg = pl.kernel(
    _deg_body,
    out_type=jax.ShapeDtypeStruct((NC, R, D), jnp.float32),
    mesh=_mesh,
    compiler_params=pltpu.CompilerParams(needs_layout_passes=False),
    scratch_types=[
        pltpu.VMEM_SHARED((AR, D), jnp.float32),
        pltpu.VMEM((AR, D), jnp.float32),
        pltpu.VMEM((DCHK,), jnp.int32),
        pltpu.VMEM((4, 8, D), jnp.float32),
        pltpu.VMEM((UNR, D), jnp.float32),
        pltpu.VMEM((C,), jnp.int32),
        pltpu.VMEM((C,), jnp.int32),
        pltpu.VMEM((AR - 2 * C,), jnp.int32),
    ],
)


BR = 400


def _embed_body(x_ref, w_ref, b_ref, dp_ref, o_ref, *, lane):
    h = jnp.dot(x_ref[...], w_ref[...], preferred_element_type=jnp.float32)
    h = h + b_ref[...]
    deg = dp_ref[0, :, lane:lane + 1] + dp_ref[1, :, lane:lane + 1]
    o_ref[...] = h * lax.rsqrt(jnp.maximum(deg, 1.0))


def _embed(x, w, b, dp, lane):
    return pl.pallas_call(
        functools.partial(_embed_body, lane=lane),
        grid=(N // BR,),
        in_specs=[
            pl.BlockSpec((BR, D), lambda i: (i, 0)),
            pl.BlockSpec((D, H), lambda i: (0, 0)),
            pl.BlockSpec((1, H), lambda i: (0, 0)),
            pl.BlockSpec((NC, BR, D), lambda i: (0, i, 0)),
        ],
        out_specs=pl.BlockSpec((BR, H), lambda i: (i, 0)),
        out_shape=jax.ShapeDtypeStruct((N, H), jnp.float32),
    )(x, w, b, dp)


def _comb_body(p_ref, d_ref, o_ref, *, lane_a, lane_b):
    t = p_ref[0, 0] + p_ref[0, 1]
    da = d_ref[0, :, lane_a:lane_a + 1] + d_ref[1, :, lane_a:lane_a + 1]
    db = d_ref[0, :, lane_b:lane_b + 1] + d_ref[1, :, lane_b:lane_b + 1]
    sa = lax.rsqrt(jnp.maximum(da, 1.0))
    sb = lax.rsqrt(jnp.maximum(db, 1.0))
    o_ref[...] = t * (sa * sb)


def _comb(p, slot, d, lane_a, lane_b):
    return pl.pallas_call(
        functools.partial(_comb_body, lane_a=lane_a, lane_b=lane_b),
        grid=(N // BR,),
        in_specs=[
            pl.BlockSpec((1, NC, BR, H), lambda i: (slot, 0, i, 0)),
            pl.BlockSpec((NC, BR, D), lambda i: (0, i, 0)),
        ],
        out_specs=pl.BlockSpec((BR, H), lambda i: (i, 0)),
        out_shape=jax.ShapeDtypeStruct((N, H), jnp.float32),
    )(p, d)


def _head_body(p_ref, d_ref, g_ref, bb_ref, w1_ref, b1_ref, w2_ref, b2_ref,
               o_ref, *, softmax, lane):
    h = p_ref[0, 0] + p_ref[0, 1]
    deg = d_ref[0, :, lane:lane + 1] + d_ref[1, :, lane:lane + 1]
    h = h * lax.rsqrt(jnp.maximum(deg, 1.0))
    mu = jnp.mean(h, axis=-1, keepdims=True)
    xc = h - mu
    var = jnp.mean(xc * xc, axis=-1, keepdims=True)
    hn = xc * lax.rsqrt(var + 1e-5) * g_ref[...] + bb_ref[...]
    z = jnp.maximum(
        jnp.dot(hn, w1_ref[...], preferred_element_type=jnp.float32)
        + b1_ref[...], 0.0)
    z = jnp.dot(z, w2_ref[...], preferred_element_type=jnp.float32) + b2_ref[...]
    if softmax:
        m = jnp.max(z, axis=-1, keepdims=True)
        z = z - m
        z = z - jnp.log(jnp.sum(jnp.exp(z), axis=-1, keepdims=True))
    o_ref[...] = z


def _head(p, slot, d, g, b, w1, b1, w2, b2, odim, softmax, lane):
    return pl.pallas_call(
        functools.partial(_head_body, softmax=softmax, lane=lane),
        grid=(N // BR,),
        in_specs=[
            pl.BlockSpec((1, NC, BR, H), lambda i: (slot, 0, i, 0)),
            pl.BlockSpec((NC, BR, D), lambda i: (0, i, 0)),
            pl.BlockSpec((1, H), lambda i: (0, 0)),
            pl.BlockSpec((1, H), lambda i: (0, 0)),
            pl.BlockSpec((H, H), lambda i: (0, 0)),
            pl.BlockSpec((1, H), lambda i: (0, 0)),
            pl.BlockSpec((H, odim), lambda i: (0, 0)),
            pl.BlockSpec((1, odim), lambda i: (0, 0)),
        ],
        out_specs=pl.BlockSpec((BR, odim), lambda i: (i, 0)),
        out_shape=jax.ShapeDtypeStruct((N, odim), jnp.float32),
    )(p, d, g, b, w1, b1, w2, b2)


def kernel(x_author, x_paper, edge_writes, edge_rev, W_emb_a, b_emb_a,
           W_emb_p, b_emb_p, ln_g_a, ln_b_a, ln_g_p, ln_b_p, Wo1_a, bo1_a,
           Wo2_a, bo2_a, Wo1_p, bo1_p, Wo2_p, bo2_p):
    ew = edge_writes.astype(jnp.int32)
    er = edge_rev.astype(jnp.int32)
    pad = E_PAD - E

    def pad_to(a, val):
        return jnp.concatenate([a, jnp.full((pad,), val, jnp.int32)])

    pad_dst = N + jnp.arange(pad, dtype=jnp.int32) % (RS - N)
    pad_src = jnp.arange(pad, dtype=jnp.int32) % N

    src_w = jnp.concatenate([ew[0], pad_src])
    dst_w = jnp.concatenate([ew[1], pad_dst])
    src_r = jnp.concatenate([er[0], pad_src])
    dst_r = jnp.concatenate([er[1], pad_dst])
    deg_idx = jnp.concatenate([
        pad_to(ew[0], N), pad_to(ew[1], N),
        pad_to(er[0], N), pad_to(er[1], N),
    ])
    zeros128 = jnp.zeros((C, D), jnp.float32)
    arange320 = jnp.arange(AR, dtype=jnp.int32)

    degp = _deg(deg_idx, zeros128, arange320)

    a0 = _embed(x_author, W_emb_a, b_emb_a.reshape(1, H), degp, 0)
    p0 = _embed(x_paper, W_emb_p, b_emb_p.reshape(1, H), degp, 2)

    tp = _spmm(a0, src_w, dst_w, zeros128)
    ta = _spmm(p0, src_r, dst_r, zeros128)

    a1 = _comb(tp, 0, degp, 1, 2)
    p1 = _comb(ta, 0, degp, 3, 0)

    hap = _spmm(a1, src_r, dst_r, zeros128)
    hpp = _spmm(p1, src_w, dst_w, zeros128)

    out_a = _head(hap, 0, degp, ln_g_a.reshape(1, H), ln_b_a.reshape(1, H),
                  Wo1_a, bo1_a.reshape(1, H), Wo2_a, bo2_a.reshape(1, OUT_A),
                  OUT_A, True, 3)
    out_p = _head(hpp, 0, degp, ln_g_p.reshape(1, H), ln_b_p.reshape(1, H),
                  Wo1_p, bo1_p.reshape(1, H), Wo2_p, bo2_p.reshape(1, H),
                  H, False, 1)
    return (out_a, out_p)


# speedup vs baseline: 1.1564x; 1.0222x over previous
import functools

import jax
import jax.numpy as jnp
from jax import lax
from jax.experimental import pallas as pl
from jax.experimental.pallas import tpu as pltpu
from jax.experimental.pallas import tpu_sc as plsc

N = 10000
D = 128
H = 128
OUT_A = 64
E = 320000

NC = 2
NS = 16
NW = NC * NS

C = 128
ECH = 80
EPW = ECH * C
E_PAD = EPW * NW

RCH = 80
R = RCH * C

_mesh = plsc.VectorSubcoreMesh(core_axis_name="c", subcore_axis_name="s")


CH0 = 80
CH1 = 80
NBUF = 3

RSCH = 79
RS = RSCH * C


def _spmm_one(x_hbm, src_hbm, dst_hbm, zeros_hbm, out_hbm, slot, acc,
              sidx, didx, rows, s_si, s_di, s_g, s_s):
    c = lax.axis_index("c")
    s = lax.axis_index("s")
    pltpu.sync_copy(zeros_hbm, rows[2])
    for j in range(5):
        ch = s * 5 + j
        @pl.when(ch < RSCH)
        def _():
            pltpu.sync_copy(rows[2], acc.at[pl.ds(ch * C, C)])
    plsc.subcore_barrier()
    kc = lax.select(c == 0, jnp.int32(CH0), jnp.int32(CH1))
    base = (c * NS * CH0 + s * kc) * C

    def issue_idx(k, b):
        off = base + k * C
        pltpu.async_copy(src_hbm.at[pl.ds(off, C)], sidx[b], s_si[b])
        pltpu.async_copy(dst_hbm.at[pl.ds(off, C)], didx[b], s_di[b])

    def wait_idx(k, b):
        off = base + k * C
        pltpu.make_async_copy(src_hbm.at[pl.ds(off, C)], sidx[b], s_si[b]).wait()
        pltpu.make_async_copy(dst_hbm.at[pl.ds(off, C)], didx[b], s_di[b]).wait()

    def issue_gather(b):
        pltpu.async_copy(x_hbm.at[sidx[b]], rows[b], s_g[b])

    def wait_gather(b):
        pltpu.make_async_copy(x_hbm.at[sidx[b]], rows[b], s_g[b]).wait()

    def issue_scatter(b):
        pltpu.async_copy(rows[b], acc.at[didx[b]], s_s[b], add=True)

    def wait_scatter(b):
        pltpu.make_async_copy(rows[b], acc.at[didx[b]], s_s[b]).wait()

    issue_idx(0, 0)
    wait_idx(0, 0)
    issue_gather(0)
    issue_idx(1, 1)
    wait_idx(1, 1)
    issue_gather(1)
    issue_idx(2, 2)
    wait_gather(0)
    issue_scatter(0)

    def triple(t, carry):
        for u in range(NBUF):
            k = 2 + NBUF * t + u
            b = (2 + u) % NBUF
            wait_idx(k, b)
            wait_scatter(u)
            issue_gather(b)
            @pl.when(k + 1 < kc)
            def _():
                issue_idx(k + 1, u)
            wait_gather((u + 1) % NBUF)
            issue_scatter((u + 1) % NBUF)
        return carry

    lax.fori_loop(0, (kc - 2) // NBUF, triple, 0)
    wait_scatter(0)
    wait_gather(1)
    issue_scatter(1)
    wait_scatter(1)
    plsc.subcore_barrier()
    for j in range(5):
        ch = s * 5 + j
        @pl.when(ch < RSCH)
        def _():
            pltpu.sync_copy(acc.at[pl.ds(ch * C, C)], rows[0])
            pltpu.sync_copy(rows[0], out_hbm.at[slot, c, pl.ds(ch * C, C)])


def _spmm_body(x_hbm, src_hbm, dst_hbm, zeros_hbm, out_hbm, acc, *bufs):
    sidx = bufs[0:3]
    didx = bufs[3:6]
    rows = bufs[6:9]
    s_si = bufs[9:12]
    s_di = bufs[12:15]
    s_g = bufs[15:18]
    s_s = bufs[18:21]
    _spmm_one(x_hbm, src_hbm, dst_hbm, zeros_hbm, out_hbm, 0, acc,
              sidx, didx, rows, s_si, s_di, s_g, s_s)


_spmm = pl.kernel(
    _spmm_body,
    out_type=jax.ShapeDtypeStruct((1, NC, RS, D), jnp.float32),
    mesh=_mesh,
    scratch_types=(
        [pltpu.VMEM_SHARED((RS, D), jnp.float32)]
        + [pltpu.VMEM((C,), jnp.int32) for _ in range(6)]
        + [pltpu.VMEM((C, D), jnp.float32) for _ in range(3)]
        + [pltpu.SemaphoreType.DMA for _ in range(12)]
    ),
)


DCHK = 1280
DGRP = DCHK // 16
HR = 80
AR = 4 * HR
UNR = 512


def _deg_body(idx_hbm, zeros_hbm, ar_hbm, out_hbm, acc, hist, idxv, pk, unp,
              ar0, ar1, ar2):
    c = lax.axis_index("c")
    s = lax.axis_index("s")
    wid = s * NC + c
    pltpu.sync_copy(zeros_hbm, hist.at[pl.ds(0, C)])
    pltpu.sync_copy(zeros_hbm, hist.at[pl.ds(C, C)])
    pltpu.sync_copy(zeros_hbm.at[pl.ds(0, AR - 2 * C)],
                    hist.at[pl.ds(2 * C, AR - 2 * C)])
    @pl.when(s < 8)
    def _():
        pltpu.sync_copy(zeros_hbm.at[pl.ds(0, 40)], acc.at[pl.ds(s * 40, 40)])
    pltpu.sync_copy(ar_hbm.at[pl.ds(0, C)], ar0)
    pltpu.sync_copy(ar_hbm.at[pl.ds(C, C)], ar1)
    pltpu.sync_copy(ar_hbm.at[pl.ds(2 * C, AR - 2 * C)], ar2)
    ones = jnp.full((16,), 1.0, jnp.float32)
    for k in range(4):
        base = k * E_PAD + wid * EPW
        def chunk(i, carry):
            pltpu.sync_copy(idx_hbm.at[pl.ds(base + i * DCHK, DCHK)], idxv)
            def grp(g, carry2):
                iv = idxv[pl.ds(g * 16, 16)]
                row = lax.shift_right_logical(iv, 7) + k * HR
                col = lax.bitwise_and(iv, 127)
                plsc.addupdate_scatter(hist, [row, col], ones)
                return carry2
            lax.fori_loop(0, DGRP, grp, 0)
            return carry
        lax.fori_loop(0, 8, chunk, 0)
    plsc.subcore_barrier()
    pltpu.sync_copy(hist.at[pl.ds(0, C)], acc.at[ar0], add=True)
    pltpu.sync_copy(hist.at[pl.ds(C, C)], acc.at[ar1], add=True)
    pltpu.sync_copy(hist.at[pl.ds(2 * C, AR - 2 * C)], acc.at[ar2], add=True)
    plsc.subcore_barrier()
    @pl.when(s < 10)
    def _():
        m = s
        for k in range(4):
            pltpu.sync_copy(acc.at[pl.ds(k * HR + 8 * m, 8)], pk.at[k])
        for h in range(2):
            for k in range(4):
                for r in range(4):
                    rr = 4 * h + r
                    for g in range(8):
                        v = pk[k, rr, pl.ds(g * 16, 16)]
                        orow = lax.iota(jnp.int32, 16) + (r * C + g * 16)
                        plsc.store_scatter(
                            unp, [orow, jnp.full((16,), k, jnp.int32)], v)
            pltpu.sync_copy(unp, out_hbm.at[c, pl.ds(1024 * m + UNR * h, UNR)])


_deg = pl.kernel(
    _deg_body,
    out_type=jax.ShapeDtypeStruct((NC, R, D), jnp.float32),
    mesh=_mesh,
    compiler_params=pltpu.CompilerParams(needs_layout_passes=False),
    scratch_types=[
        pltpu.VMEM_SHARED((AR, D), jnp.float32),
        pltpu.VMEM((AR, D), jnp.float32),
        pltpu.VMEM((DCHK,), jnp.int32),
        pltpu.VMEM((4, 8, D), jnp.float32),
        pltpu.VMEM((UNR, D), jnp.float32),
        pltpu.VMEM((C,), jnp.int32),
        pltpu.VMEM((C,), jnp.int32),
        pltpu.VMEM((AR - 2 * C,), jnp.int32),
    ],
)


BR = 400


def _embed_body(x_ref, w_ref, b_ref, dp_ref, o_ref, *, lane):
    h = jnp.dot(x_ref[...], w_ref[...], preferred_element_type=jnp.float32)
    h = h + b_ref[...]
    deg = dp_ref[0, :, lane:lane + 1] + dp_ref[1, :, lane:lane + 1]
    o_ref[...] = h * lax.rsqrt(jnp.maximum(deg, 1.0))


def _embed(x, w, b, dp, lane):
    return pl.pallas_call(
        functools.partial(_embed_body, lane=lane),
        grid=(N // BR,),
        in_specs=[
            pl.BlockSpec((BR, D), lambda i: (i, 0)),
            pl.BlockSpec((D, H), lambda i: (0, 0)),
            pl.BlockSpec((1, H), lambda i: (0, 0)),
            pl.BlockSpec((NC, BR, D), lambda i: (0, i, 0)),
        ],
        out_specs=pl.BlockSpec((BR, H), lambda i: (i, 0)),
        out_shape=jax.ShapeDtypeStruct((N, H), jnp.float32),
    )(x, w, b, dp)


def _comb_body(p_ref, d_ref, o_ref, *, lane_a, lane_b):
    t = p_ref[0, 0] + p_ref[0, 1]
    da = d_ref[0, :, lane_a:lane_a + 1] + d_ref[1, :, lane_a:lane_a + 1]
    db = d_ref[0, :, lane_b:lane_b + 1] + d_ref[1, :, lane_b:lane_b + 1]
    sa = lax.rsqrt(jnp.maximum(da, 1.0))
    sb = lax.rsqrt(jnp.maximum(db, 1.0))
    o_ref[...] = t * (sa * sb)


def _comb(p, slot, d, lane_a, lane_b):
    return pl.pallas_call(
        functools.partial(_comb_body, lane_a=lane_a, lane_b=lane_b),
        grid=(N // BR,),
        in_specs=[
            pl.BlockSpec((1, NC, BR, H), lambda i: (slot, 0, i, 0)),
            pl.BlockSpec((NC, BR, D), lambda i: (0, i, 0)),
        ],
        out_specs=pl.BlockSpec((BR, H), lambda i: (i, 0)),
        out_shape=jax.ShapeDtypeStruct((N, H), jnp.float32),
    )(p, d)


def _head_body(p_ref, d_ref, g_ref, bb_ref, w1_ref, b1_ref, w2_ref, b2_ref,
               o_ref, *, softmax, lane):
    h = p_ref[0, 0] + p_ref[0, 1]
    deg = d_ref[0, :, lane:lane + 1] + d_ref[1, :, lane:lane + 1]
    h = h * lax.rsqrt(jnp.maximum(deg, 1.0))
    mu = jnp.mean(h, axis=-1, keepdims=True)
    xc = h - mu
    var = jnp.mean(xc * xc, axis=-1, keepdims=True)
    hn = xc * lax.rsqrt(var + 1e-5) * g_ref[...] + bb_ref[...]
    z = jnp.maximum(
        jnp.dot(hn, w1_ref[...], preferred_element_type=jnp.float32)
        + b1_ref[...], 0.0)
    z = jnp.dot(z, w2_ref[...], preferred_element_type=jnp.float32) + b2_ref[...]
    if softmax:
        m = jnp.max(z, axis=-1, keepdims=True)
        z = z - m
        z = z - jnp.log(jnp.sum(jnp.exp(z), axis=-1, keepdims=True))
    o_ref[...] = z


def _head(p, slot, d, g, b, w1, b1, w2, b2, odim, softmax, lane):
    return pl.pallas_call(
        functools.partial(_head_body, softmax=softmax, lane=lane),
        grid=(N // BR,),
        in_specs=[
            pl.BlockSpec((1, NC, BR, H), lambda i: (slot, 0, i, 0)),
            pl.BlockSpec((NC, BR, D), lambda i: (0, i, 0)),
            pl.BlockSpec((1, H), lambda i: (0, 0)),
            pl.BlockSpec((1, H), lambda i: (0, 0)),
            pl.BlockSpec((H, H), lambda i: (0, 0)),
            pl.BlockSpec((1, H), lambda i: (0, 0)),
            pl.BlockSpec((H, odim), lambda i: (0, 0)),
            pl.BlockSpec((1, odim), lambda i: (0, 0)),
        ],
        out_specs=pl.BlockSpec((BR, odim), lambda i: (i, 0)),
        out_shape=jax.ShapeDtypeStruct((N, odim), jnp.float32),
    )(p, d, g, b, w1, b1, w2, b2)


def kernel(x_author, x_paper, edge_writes, edge_rev, W_emb_a, b_emb_a,
           W_emb_p, b_emb_p, ln_g_a, ln_b_a, ln_g_p, ln_b_p, Wo1_a, bo1_a,
           Wo2_a, bo2_a, Wo1_p, bo1_p, Wo2_p, bo2_p):
    ew = edge_writes.astype(jnp.int32)
    er = edge_rev.astype(jnp.int32)
    pad = E_PAD - E

    def pad_to(a, val):
        return jnp.concatenate([a, jnp.full((pad,), val, jnp.int32)])

    pad_dst = N + jnp.arange(pad, dtype=jnp.int32) % (RS - N)
    pad_src = jnp.arange(pad, dtype=jnp.int32) % N

    src_w = jnp.concatenate([ew[0], pad_src])
    dst_w = jnp.concatenate([ew[1], pad_dst])
    src_r = jnp.concatenate([er[0], pad_src])
    dst_r = jnp.concatenate([er[1], pad_dst])
    deg_idx = jnp.concatenate([
        pad_to(ew[0], N), pad_to(ew[1], N),
        pad_to(er[0], N), pad_to(er[1], N),
    ])
    zeros128 = jnp.zeros((C, D), jnp.float32)
    arange320 = jnp.arange(AR, dtype=jnp.int32)

    degp = _deg(deg_idx, zeros128, arange320)

    a0 = _embed(x_author, W_emb_a, b_emb_a.reshape(1, H), degp, 0)
    p0 = _embed(x_paper, W_emb_p, b_emb_p.reshape(1, H), degp, 2)

    tp = _spmm(a0, src_w, dst_w, zeros128)
    ta = _spmm(p0, src_r, dst_r, zeros128)

    a1 = _comb(tp, 0, degp, 1, 2)
    p1 = _comb(ta, 0, degp, 3, 0)

    hap = _spmm(a1, src_r, dst_r, zeros128)
    hpp = _spmm(p1, src_w, dst_w, zeros128)

    out_a = _head(hap, 0, degp, ln_g_a.reshape(1, H), ln_b_a.reshape(1, H),
                  Wo1_a, bo1_a.reshape(1, H), Wo2_a, bo2_a.reshape(1, OUT_A),
                  OUT_A, True, 3)
    out_p = _head(hpp, 0, degp, ln_g_p.reshape(1, H), ln_b_p.reshape(1, H),
                  Wo1_p, bo1_p.reshape(1, H), Wo2_p, bo2_p.reshape(1, H),
                  H, False, 1)
    return (out_a, out_p)
